# baseline (speedup 1.0000x reference)
import jax
import jax.numpy as jnp
from jax import lax
from jax.experimental import pallas as pl
from jax.experimental.pallas import tpu as pltpu
from jax.experimental.pallas import tpu_sc as plsc

T = 2048
D = 1024
E = 8
K = 2
A = K * T
B = 256
NPAD = A + E * B
NB = NPAD // B
NEG_INF = -1e30

NUM_TILES = 32
TOK_PER_TILE = T // NUM_TILES
CCH = 32


def _routing_body(x_ref, wg_ref, dest_ref, w_ref, eid_ref):
    logits = jax.lax.dot_general(
        wg_ref[...], x_ref[...], (((1,), (1,)), ((), ())),
        preferred_element_type=jnp.float32)
    sub = jax.lax.broadcasted_iota(jnp.int32, (E, T), 0)
    m1 = jnp.max(logits, axis=0, keepdims=True)
    a1 = jnp.min(jnp.where(logits == m1, sub, E), axis=0, keepdims=True)
    masked = jnp.where(sub == a1, NEG_INF, logits)
    m2 = jnp.max(masked, axis=0, keepdims=True)
    a2 = jnp.min(jnp.where(masked == m2, sub, E), axis=0, keepdims=True)
    w1 = 1.0 / (1.0 + jnp.exp(m2 - m1))
    w2 = 1.0 - w1

    ind = (jnp.where(sub == a1, 1.0, 0.0)
           + jnp.where(sub == a2, 1.0, 0.0))
    r = jax.lax.broadcasted_iota(jnp.int32, (T, T), 0)
    c = jax.lax.broadcasted_iota(jnp.int32, (T, T), 1)
    ut = jnp.where(r < c, 1.0, 0.0)
    rank = jax.lax.dot_general(
        ind, ut, (((1,), (0,)), ((), ())),
        preferred_element_type=jnp.float32)
    counts = jnp.sum(ind, axis=1, keepdims=True).astype(jnp.int32)
    padded = ((counts + (B - 1)) // B) * B
    esub = jax.lax.broadcasted_iota(jnp.int32, (E, E), 0)
    ecol = jax.lax.broadcasted_iota(jnp.int32, (E, E), 1)
    ltri = jnp.where(ecol < esub, 1.0, 0.0)
    pstart = jax.lax.dot_general(
        ltri, padded.astype(jnp.float32), (((1,), (0,)), ((), ())),
        preferred_element_type=jnp.float32)

    slot = rank + pstart
    d1 = jnp.sum(jnp.where(sub == a1, slot, 0.0), axis=0, keepdims=True)
    d2 = jnp.sum(jnp.where(sub == a2, slot, 0.0), axis=0, keepdims=True)
    dest_ref[0:1, :] = d1.astype(jnp.int32)
    dest_ref[1:2, :] = d2.astype(jnp.int32)
    w_ref[0:1, :] = w1
    w_ref[1:2, :] = w2

    bstart = jax.lax.broadcasted_iota(jnp.int32, (E, NB), 1) * B
    ge = jnp.where(bstart >= pstart.astype(jnp.int32), 1, 0)
    eid_ref[...] = jnp.sum(ge, axis=0, keepdims=True) - 1


def _routing(x, Wg):
    return pl.pallas_call(
        _routing_body,
        grid=(1,),
        in_specs=[
            pl.BlockSpec((T, D), lambda i: (0, 0)),
            pl.BlockSpec((E, D), lambda i: (0, 0)),
        ],
        out_specs=[
            pl.BlockSpec((K, T), lambda i: (0, 0)),
            pl.BlockSpec((K, T), lambda i: (0, 0)),
            pl.BlockSpec((1, NB), lambda i: (0, 0)),
        ],
        out_shape=[
            jax.ShapeDtypeStruct((K, T), jnp.int32),
            jax.ShapeDtypeStruct((K, T), jnp.float32),
            jax.ShapeDtypeStruct((1, NB), jnp.int32),
        ],
    )(x, Wg)


def _dispatch_body(x_hbm, dest_hbm, xs_hbm, idx0, idx1, rows, sem0, sem1):
    wid = lax.axis_index("s") * 2 + lax.axis_index("c")
    tb = wid * TOK_PER_TILE
    pltpu.sync_copy(dest_hbm.at[0, pl.ds(tb, TOK_PER_TILE)], idx0)
    pltpu.sync_copy(dest_hbm.at[1, pl.ds(tb, TOK_PER_TILE)], idx1)
    pltpu.sync_copy(x_hbm.at[pl.ds(tb, TOK_PER_TILE)], rows)
    c0 = pltpu.async_copy(rows, xs_hbm.at[idx0], sem0)
    c1 = pltpu.async_copy(rows, xs_hbm.at[idx1], sem1)
    c0.wait()
    c1.wait()


def _dispatch(x, dest):
    f = pl.kernel(
        _dispatch_body,
        out_type=jax.ShapeDtypeStruct((NPAD, D), jnp.float32),
        mesh=plsc.VectorSubcoreMesh(core_axis_name="c", subcore_axis_name="s"),
        compiler_params=pltpu.CompilerParams(needs_layout_passes=False),
        scratch_types=[
            pltpu.VMEM((TOK_PER_TILE,), jnp.int32),
            pltpu.VMEM((TOK_PER_TILE,), jnp.int32),
            pltpu.VMEM((TOK_PER_TILE, D), jnp.float32),
            pltpu.SemaphoreType.DMA,
            pltpu.SemaphoreType.DMA,
        ],
    )
    return f(x, dest)


def _expert_body(eid_ref, xs_ref, we_hbm, be_ref, ys_ref,
                 we_f32, we_bf16, sem):
    i = pl.program_id(0)
    e = eid_ref[i]
    prev = eid_ref[jnp.maximum(i - 1, 0)]

    @pl.when(jnp.logical_or(i == 0, e != prev))
    def _fetch():
        copy = pltpu.make_async_copy(we_hbm.at[e], we_f32, sem)
        copy.start()
        copy.wait()
        we_bf16[...] = we_f32[...].astype(jnp.bfloat16)

    y = jax.lax.dot_general(
        xs_ref[...].astype(jnp.bfloat16),
        we_bf16[...],
        (((1,), (1,)), ((), ())),
        preferred_element_type=jnp.float32)
    ys_ref[...] = y + be_ref[0]


def _expert_matmul(eid, xs, We, be):
    grid_spec = pltpu.PrefetchScalarGridSpec(
        num_scalar_prefetch=1,
        grid=(NB,),
        in_specs=[
            pl.BlockSpec((B, D), lambda i, eid: (i, 0)),
            pl.BlockSpec(memory_space=pl.ANY),
            pl.BlockSpec((1, 1, D), lambda i, eid: (eid[i], 0, 0)),
        ],
        out_specs=pl.BlockSpec((B, D), lambda i, eid: (i, 0)),
        scratch_shapes=[
            pltpu.VMEM((D, D), jnp.float32),
            pltpu.VMEM((D, D), jnp.bfloat16),
            pltpu.SemaphoreType.DMA,
        ],
    )
    return pl.pallas_call(
        _expert_body,
        grid_spec=grid_spec,
        out_shape=jax.ShapeDtypeStruct((NPAD, D), jnp.float32),
    )(eid, xs, We, be.reshape(E, 1, D))


def _combine_body(ys_hbm, dest_hbm, w_hbm, out_hbm,
                  d0, d1, wv0, wv1, buf0, buf1, sem0, sem1):
    wid = lax.axis_index("s") * 2 + lax.axis_index("c")
    tb = wid * TOK_PER_TILE
    pltpu.sync_copy(dest_hbm.at[0, pl.ds(tb, TOK_PER_TILE)], d0)
    pltpu.sync_copy(dest_hbm.at[1, pl.ds(tb, TOK_PER_TILE)], d1)
    pltpu.sync_copy(w_hbm.at[0, pl.ds(tb, TOK_PER_TILE)], wv0)
    pltpu.sync_copy(w_hbm.at[1, pl.ds(tb, TOK_PER_TILE)], wv1)
    for ch in range(TOK_PER_TILE // CCH):
        c0 = pltpu.async_copy(
            ys_hbm.at[d0.at[pl.ds(ch * CCH, CCH)]], buf0, sem0)
        c1 = pltpu.async_copy(
            ys_hbm.at[d1.at[pl.ds(ch * CCH, CCH)]], buf1, sem1)
        c0.wait()
        c1.wait()

        def tok_body(t, _):
            tsplat = jnp.zeros((16,), jnp.int32) + (ch * CCH + t)
            w1v = plsc.load_gather(wv0, [tsplat])
            w2v = plsc.load_gather(wv1, [tsplat])

            def col_body(i, _):
                buf0[t, pl.ds(i * 16, 16)] = (
                    w1v * buf0[t, pl.ds(i * 16, 16)]
                    + w2v * buf1[t, pl.ds(i * 16, 16)])
                return 0

            lax.fori_loop(0, D // 16, col_body, 0, unroll=8)
            return 0

        lax.fori_loop(0, CCH, tok_body, 0)
        pltpu.sync_copy(buf0, out_hbm.at[pl.ds(tb + ch * CCH, CCH)])


def _combine(ys, dest, w):
    f = pl.kernel(
        _combine_body,
        out_type=jax.ShapeDtypeStruct((T, D), jnp.float32),
        mesh=plsc.VectorSubcoreMesh(core_axis_name="c", subcore_axis_name="s"),
        compiler_params=pltpu.CompilerParams(needs_layout_passes=False),
        scratch_types=[
            pltpu.VMEM((TOK_PER_TILE,), jnp.int32),
            pltpu.VMEM((TOK_PER_TILE,), jnp.int32),
            pltpu.VMEM((TOK_PER_TILE,), jnp.float32),
            pltpu.VMEM((TOK_PER_TILE,), jnp.float32),
            pltpu.VMEM((CCH, D), jnp.float32),
            pltpu.VMEM((CCH, D), jnp.float32),
            pltpu.SemaphoreType.DMA,
            pltpu.SemaphoreType.DMA,
        ],
    )
    return f(ys, dest, w)


def kernel(inputs, Wg, We, be):
    dest, w, eid = _routing(inputs, Wg)
    xs = _dispatch(inputs, dest)
    ys = _expert_matmul(eid.reshape(NB), xs, We, be)
    return _combine(ys, dest, w)


# speedup vs baseline: 1.0463x; 1.0463x over previous
import jax
import jax.numpy as jnp
from jax import lax
from jax.experimental import pallas as pl
from jax.experimental.pallas import tpu as pltpu
from jax.experimental.pallas import tpu_sc as plsc

T = 2048
D = 1024
E = 8
K = 2
A = K * T
B = 256
NPAD = A + E * B
NB = NPAD // B
NEG_INF = -1e30

NUM_TILES = 32
TOK_PER_TILE = T // NUM_TILES
CCH = 32


def _routing_body(x_ref, wg_ref, dest_ref, w1x_ref, w2x_ref, eid_ref):
    logits = jax.lax.dot_general(
        wg_ref[...], x_ref[...], (((1,), (1,)), ((), ())),
        preferred_element_type=jnp.float32)
    sub = jax.lax.broadcasted_iota(jnp.int32, (E, T), 0)
    m1 = jnp.max(logits, axis=0, keepdims=True)
    a1 = jnp.min(jnp.where(logits == m1, sub, E), axis=0, keepdims=True)
    masked = jnp.where(sub == a1, NEG_INF, logits)
    m2 = jnp.max(masked, axis=0, keepdims=True)
    a2 = jnp.min(jnp.where(masked == m2, sub, E), axis=0, keepdims=True)
    w1 = 1.0 / (1.0 + jnp.exp(m2 - m1))
    w2 = 1.0 - w1

    ind = (jnp.where(sub == a1, 1.0, 0.0)
           + jnp.where(sub == a2, 1.0, 0.0))
    r = jax.lax.broadcasted_iota(jnp.int32, (T, T), 0)
    c = jax.lax.broadcasted_iota(jnp.int32, (T, T), 1)
    ut = jnp.where(r < c, 1.0, 0.0)
    rank = jax.lax.dot_general(
        ind, ut, (((1,), (0,)), ((), ())),
        preferred_element_type=jnp.float32)
    counts = jnp.sum(ind, axis=1, keepdims=True).astype(jnp.int32)
    padded = ((counts + (B - 1)) // B) * B
    esub = jax.lax.broadcasted_iota(jnp.int32, (E, E), 0)
    ecol = jax.lax.broadcasted_iota(jnp.int32, (E, E), 1)
    ltri = jnp.where(ecol < esub, 1.0, 0.0)
    pstart = jax.lax.dot_general(
        ltri, padded.astype(jnp.float32), (((1,), (0,)), ((), ())),
        preferred_element_type=jnp.float32)

    slot = rank + pstart
    d1 = jnp.sum(jnp.where(sub == a1, slot, 0.0), axis=0, keepdims=True)
    d2 = jnp.sum(jnp.where(sub == a2, slot, 0.0), axis=0, keepdims=True)
    dest_ref[0:1, :] = d1.astype(jnp.int32)
    dest_ref[1:2, :] = d2.astype(jnp.int32)

    ident = jnp.where(r == c, 1.0, 0.0)
    wrows = jnp.concatenate([w1, w2], axis=0)
    wcol = jax.lax.dot_general(
        ident, wrows, (((1,), (1,)), ((), ())),
        preferred_element_type=jnp.float32)
    w1x_ref[...] = jax.lax.broadcast_in_dim(wcol[:, 0:1], (T, 16), (0, 1))
    w2x_ref[...] = jax.lax.broadcast_in_dim(wcol[:, 1:2], (T, 16), (0, 1))

    bstart = jax.lax.broadcasted_iota(jnp.int32, (E, NB), 1) * B
    ge = jnp.where(bstart >= pstart.astype(jnp.int32), 1, 0)
    eid_ref[...] = jnp.sum(ge, axis=0, keepdims=True) - 1


def _routing(x, Wg):
    return pl.pallas_call(
        _routing_body,
        grid=(1,),
        in_specs=[
            pl.BlockSpec((T, D), lambda i: (0, 0)),
            pl.BlockSpec((E, D), lambda i: (0, 0)),
        ],
        out_specs=[
            pl.BlockSpec((K, T), lambda i: (0, 0)),
            pl.BlockSpec((T, 16), lambda i: (0, 0)),
            pl.BlockSpec((T, 16), lambda i: (0, 0)),
            pl.BlockSpec((1, NB), lambda i: (0, 0)),
        ],
        out_shape=[
            jax.ShapeDtypeStruct((K, T), jnp.int32),
            jax.ShapeDtypeStruct((T, 16), jnp.float32),
            jax.ShapeDtypeStruct((T, 16), jnp.float32),
            jax.ShapeDtypeStruct((1, NB), jnp.int32),
        ],
    )(x, Wg)


def _dispatch_body(x_hbm, dest_hbm, xs_hbm, idx0, idx1, rows, sem0, sem1):
    wid = lax.axis_index("s") * 2 + lax.axis_index("c")
    tb = wid * TOK_PER_TILE
    pltpu.sync_copy(dest_hbm.at[0, pl.ds(tb, TOK_PER_TILE)], idx0)
    pltpu.sync_copy(dest_hbm.at[1, pl.ds(tb, TOK_PER_TILE)], idx1)
    pltpu.sync_copy(x_hbm.at[pl.ds(tb, TOK_PER_TILE)], rows)
    c0 = pltpu.async_copy(rows, xs_hbm.at[idx0], sem0)
    c1 = pltpu.async_copy(rows, xs_hbm.at[idx1], sem1)
    c0.wait()
    c1.wait()


def _dispatch(x, dest):
    f = pl.kernel(
        _dispatch_body,
        out_type=jax.ShapeDtypeStruct((NPAD, D), jnp.float32),
        mesh=plsc.VectorSubcoreMesh(core_axis_name="c", subcore_axis_name="s"),
        compiler_params=pltpu.CompilerParams(needs_layout_passes=False),
        scratch_types=[
            pltpu.VMEM((TOK_PER_TILE,), jnp.int32),
            pltpu.VMEM((TOK_PER_TILE,), jnp.int32),
            pltpu.VMEM((TOK_PER_TILE, D), jnp.float32),
            pltpu.SemaphoreType.DMA,
            pltpu.SemaphoreType.DMA,
        ],
    )
    return f(x, dest)


def _expert_body(eid_ref, xs_ref, we_ref, be_ref, ys_ref):
    y = jax.lax.dot_general(
        xs_ref[...].astype(jnp.bfloat16),
        we_ref[0].astype(jnp.bfloat16),
        (((1,), (1,)), ((), ())),
        preferred_element_type=jnp.float32)
    ys_ref[...] = y + be_ref[0]


def _expert_matmul(eid, xs, We, be):
    grid_spec = pltpu.PrefetchScalarGridSpec(
        num_scalar_prefetch=1,
        grid=(NB,),
        in_specs=[
            pl.BlockSpec((B, D), lambda i, eid: (i, 0)),
            pl.BlockSpec((1, D, D), lambda i, eid: (eid[i], 0, 0)),
            pl.BlockSpec((1, 1, D), lambda i, eid: (eid[i], 0, 0)),
        ],
        out_specs=pl.BlockSpec((B, D), lambda i, eid: (i, 0)),
    )
    return pl.pallas_call(
        _expert_body,
        grid_spec=grid_spec,
        out_shape=jax.ShapeDtypeStruct((NPAD, D), jnp.float32),
    )(eid, xs, We, be.reshape(E, 1, D))


def _combine_body(ys_hbm, dest_hbm, w1x_hbm, w2x_hbm, out_hbm,
                  d0, d1, wv1, wv2, buf0, buf1, sem0, sem1):
    wid = lax.axis_index("s") * 2 + lax.axis_index("c")
    tb = wid * TOK_PER_TILE
    pltpu.sync_copy(dest_hbm.at[0, pl.ds(tb, TOK_PER_TILE)], d0)
    pltpu.sync_copy(dest_hbm.at[1, pl.ds(tb, TOK_PER_TILE)], d1)
    pltpu.sync_copy(w1x_hbm.at[pl.ds(tb, TOK_PER_TILE)], wv1)
    pltpu.sync_copy(w2x_hbm.at[pl.ds(tb, TOK_PER_TILE)], wv2)
    for ch in range(TOK_PER_TILE // CCH):
        c0 = pltpu.async_copy(
            ys_hbm.at[d0.at[pl.ds(ch * CCH, CCH)]], buf0, sem0)
        c1 = pltpu.async_copy(
            ys_hbm.at[d1.at[pl.ds(ch * CCH, CCH)]], buf1, sem1)
        c0.wait()
        c1.wait()

        def tok_body(t, _):
            w1v = wv1[ch * CCH + t, :]
            w2v = wv2[ch * CCH + t, :]

            def col_body(i, _):
                buf0[t, pl.ds(i * 16, 16)] = (
                    w1v * buf0[t, pl.ds(i * 16, 16)]
                    + w2v * buf1[t, pl.ds(i * 16, 16)])
                return 0

            lax.fori_loop(0, D // 16, col_body, 0, unroll=8)
            return 0

        lax.fori_loop(0, CCH, tok_body, 0)
        pltpu.sync_copy(buf0, out_hbm.at[pl.ds(tb + ch * CCH, CCH)])


def _combine(ys, dest, w1x, w2x):
    f = pl.kernel(
        _combine_body,
        out_type=jax.ShapeDtypeStruct((T, D), jnp.float32),
        mesh=plsc.VectorSubcoreMesh(core_axis_name="c", subcore_axis_name="s"),
        compiler_params=pltpu.CompilerParams(needs_layout_passes=False),
        scratch_types=[
            pltpu.VMEM((TOK_PER_TILE,), jnp.int32),
            pltpu.VMEM((TOK_PER_TILE,), jnp.int32),
            pltpu.VMEM((TOK_PER_TILE, 16), jnp.float32),
            pltpu.VMEM((TOK_PER_TILE, 16), jnp.float32),
            pltpu.VMEM((CCH, D), jnp.float32),
            pltpu.VMEM((CCH, D), jnp.float32),
            pltpu.SemaphoreType.DMA,
            pltpu.SemaphoreType.DMA,
        ],
    )
    return f(ys, dest, w1x, w2x)


def kernel(inputs, Wg, We, be):
    dest, w1x, w2x, eid = _routing(inputs, Wg)
    xs = _dispatch(inputs, dest)
    ys = _expert_matmul(eid.reshape(NB), xs, We, be)
    return _combine(ys, dest, w1x, w2x)


# speedup vs baseline: 1.2407x; 1.1857x over previous
import jax
import jax.numpy as jnp
from jax import lax
from jax.experimental import pallas as pl
from jax.experimental.pallas import tpu as pltpu
from jax.experimental.pallas import tpu_sc as plsc

T = 2048
D = 1024
E = 8
K = 2
A = K * T
B = 256
NPAD = A + E * B
NB = NPAD // B
NEG_INF = -1e30

NUM_TILES = 32
TOK_PER_TILE = T // NUM_TILES
CCH = 32


def _routing_body(x_ref, wg_ref, dest_ref, w1x_ref, w2x_ref, eid_ref):
    logits = jax.lax.dot_general(
        wg_ref[...], x_ref[...], (((1,), (1,)), ((), ())),
        preferred_element_type=jnp.float32)
    sub = jax.lax.broadcasted_iota(jnp.int32, (E, T), 0)
    m1 = jnp.max(logits, axis=0, keepdims=True)
    a1 = jnp.min(jnp.where(logits == m1, sub, E), axis=0, keepdims=True)
    masked = jnp.where(sub == a1, NEG_INF, logits)
    m2 = jnp.max(masked, axis=0, keepdims=True)
    a2 = jnp.min(jnp.where(masked == m2, sub, E), axis=0, keepdims=True)
    w1 = 1.0 / (1.0 + jnp.exp(m2 - m1))
    w2 = 1.0 - w1

    ind = (jnp.where(sub == a1, 1.0, 0.0)
           + jnp.where(sub == a2, 1.0, 0.0))
    r = jax.lax.broadcasted_iota(jnp.int32, (T, T), 0)
    c = jax.lax.broadcasted_iota(jnp.int32, (T, T), 1)
    ut = jnp.where(r < c, 1.0, 0.0)
    rank = jax.lax.dot_general(
        ind, ut, (((1,), (0,)), ((), ())),
        preferred_element_type=jnp.float32)
    counts = jnp.sum(ind, axis=1, keepdims=True).astype(jnp.int32)
    padded = ((counts + (B - 1)) // B) * B
    esub = jax.lax.broadcasted_iota(jnp.int32, (E, E), 0)
    ecol = jax.lax.broadcasted_iota(jnp.int32, (E, E), 1)
    ltri = jnp.where(ecol < esub, 1.0, 0.0)
    pstart = jax.lax.dot_general(
        ltri, padded.astype(jnp.float32), (((1,), (0,)), ((), ())),
        preferred_element_type=jnp.float32)

    slot = rank + pstart
    d1 = jnp.sum(jnp.where(sub == a1, slot, 0.0), axis=0, keepdims=True)
    d2 = jnp.sum(jnp.where(sub == a2, slot, 0.0), axis=0, keepdims=True)
    dest_ref[0:1, :] = d1.astype(jnp.int32)
    dest_ref[1:2, :] = d2.astype(jnp.int32)

    ident = jnp.where(r == c, 1.0, 0.0)
    wrows = jnp.concatenate([w1, w2], axis=0)
    wcol = jax.lax.dot_general(
        ident, wrows, (((1,), (1,)), ((), ())),
        preferred_element_type=jnp.float32)
    w1x_ref[...] = jax.lax.broadcast_in_dim(wcol[:, 0:1], (T, 16), (0, 1))
    w2x_ref[...] = jax.lax.broadcast_in_dim(wcol[:, 1:2], (T, 16), (0, 1))

    bstart = jax.lax.broadcasted_iota(jnp.int32, (E, NB), 1) * B
    ge = jnp.where(bstart >= pstart.astype(jnp.int32), 1, 0)
    eid_ref[...] = jnp.sum(ge, axis=0, keepdims=True) - 1


def _routing(x, Wg):
    return pl.pallas_call(
        _routing_body,
        grid=(1,),
        in_specs=[
            pl.BlockSpec((T, D), lambda i: (0, 0)),
            pl.BlockSpec((E, D), lambda i: (0, 0)),
        ],
        out_specs=[
            pl.BlockSpec((K, T), lambda i: (0, 0)),
            pl.BlockSpec((T, 16), lambda i: (0, 0)),
            pl.BlockSpec((T, 16), lambda i: (0, 0)),
            pl.BlockSpec((1, NB), lambda i: (0, 0)),
        ],
        out_shape=[
            jax.ShapeDtypeStruct((K, T), jnp.int32),
            jax.ShapeDtypeStruct((T, 16), jnp.float32),
            jax.ShapeDtypeStruct((T, 16), jnp.float32),
            jax.ShapeDtypeStruct((1, NB), jnp.int32),
        ],
    )(x, Wg)


def _dispatch_body(x_hbm, dest_hbm, xs_hbm, idx0, idx1, rows, sem0, sem1):
    wid = lax.axis_index("s") * 2 + lax.axis_index("c")
    tb = wid * TOK_PER_TILE
    pltpu.sync_copy(dest_hbm.at[0, pl.ds(tb, TOK_PER_TILE)], idx0)
    pltpu.sync_copy(dest_hbm.at[1, pl.ds(tb, TOK_PER_TILE)], idx1)
    pltpu.sync_copy(x_hbm.at[pl.ds(tb, TOK_PER_TILE)], rows)
    c0 = pltpu.async_copy(rows, xs_hbm.at[idx0], sem0)
    c1 = pltpu.async_copy(rows, xs_hbm.at[idx1], sem1)
    c0.wait()
    c1.wait()


def _dispatch(x, dest):
    f = pl.kernel(
        _dispatch_body,
        out_type=jax.ShapeDtypeStruct((NPAD, D), jnp.float32),
        mesh=plsc.VectorSubcoreMesh(core_axis_name="c", subcore_axis_name="s"),
        compiler_params=pltpu.CompilerParams(needs_layout_passes=False),
        scratch_types=[
            pltpu.VMEM((TOK_PER_TILE,), jnp.int32),
            pltpu.VMEM((TOK_PER_TILE,), jnp.int32),
            pltpu.VMEM((TOK_PER_TILE, D), jnp.float32),
            pltpu.SemaphoreType.DMA,
            pltpu.SemaphoreType.DMA,
        ],
    )
    return f(x, dest)


def _expert_body(eid_ref, xs_ref, we_ref, be_ref, ys_ref):
    y = jax.lax.dot_general(
        xs_ref[...].astype(jnp.bfloat16),
        we_ref[0].astype(jnp.bfloat16),
        (((1,), (1,)), ((), ())),
        preferred_element_type=jnp.float32)
    ys_ref[...] = y + be_ref[0]


def _expert_matmul(eid, xs, We, be):
    grid_spec = pltpu.PrefetchScalarGridSpec(
        num_scalar_prefetch=1,
        grid=(NB,),
        in_specs=[
            pl.BlockSpec((B, D), lambda i, eid: (i, 0)),
            pl.BlockSpec((1, D, D), lambda i, eid: (eid[i], 0, 0)),
            pl.BlockSpec((1, 1, D), lambda i, eid: (eid[i], 0, 0)),
        ],
        out_specs=pl.BlockSpec((B, D), lambda i, eid: (i, 0)),
    )
    return pl.pallas_call(
        _expert_body,
        grid_spec=grid_spec,
        out_shape=jax.ShapeDtypeStruct((NPAD, D), jnp.float32),
    )(eid, xs, We, be.reshape(E, 1, D))


def _combine_body(ys_hbm, dest_hbm, w1x_hbm, w2x_hbm, out_hbm,
                  d0, d1, wv1, wv2, buf0, buf1, sem0, sem1):
    wid = lax.axis_index("s") * 2 + lax.axis_index("c")
    tb = wid * TOK_PER_TILE
    pltpu.sync_copy(dest_hbm.at[0, pl.ds(tb, TOK_PER_TILE)], d0)
    pltpu.sync_copy(dest_hbm.at[1, pl.ds(tb, TOK_PER_TILE)], d1)
    pltpu.sync_copy(w1x_hbm.at[pl.ds(tb, TOK_PER_TILE)], wv1)
    pltpu.sync_copy(w2x_hbm.at[pl.ds(tb, TOK_PER_TILE)], wv2)
    for ch in range(TOK_PER_TILE // CCH):
        c0 = pltpu.async_copy(
            ys_hbm.at[d0.at[pl.ds(ch * CCH, CCH)]], buf0, sem0)
        c1 = pltpu.async_copy(
            ys_hbm.at[d1.at[pl.ds(ch * CCH, CCH)]], buf1, sem1)
        c0.wait()
        c1.wait()

        @plsc.parallel_loop(0, CCH * (D // 16), step=1, unroll=8)
        def _fma(j):
            t = j >> 6
            i = j & (D // 16 - 1)
            w1v = wv1[ch * CCH + t, :]
            w2v = wv2[ch * CCH + t, :]
            buf0[t, pl.ds(i * 16, 16)] = (
                w1v * buf0[t, pl.ds(i * 16, 16)]
                + w2v * buf1[t, pl.ds(i * 16, 16)])

        pltpu.sync_copy(buf0, out_hbm.at[pl.ds(tb + ch * CCH, CCH)])


def _combine(ys, dest, w1x, w2x):
    f = pl.kernel(
        _combine_body,
        out_type=jax.ShapeDtypeStruct((T, D), jnp.float32),
        mesh=plsc.VectorSubcoreMesh(core_axis_name="c", subcore_axis_name="s"),
        compiler_params=pltpu.CompilerParams(needs_layout_passes=False),
        scratch_types=[
            pltpu.VMEM((TOK_PER_TILE,), jnp.int32),
            pltpu.VMEM((TOK_PER_TILE,), jnp.int32),
            pltpu.VMEM((TOK_PER_TILE, 16), jnp.float32),
            pltpu.VMEM((TOK_PER_TILE, 16), jnp.float32),
            pltpu.VMEM((CCH, D), jnp.float32),
            pltpu.VMEM((CCH, D), jnp.float32),
            pltpu.SemaphoreType.DMA,
            pltpu.SemaphoreType.DMA,
        ],
    )
    return f(ys, dest, w1x, w2x)


def kernel(inputs, Wg, We, be):
    dest, w1x, w2x, eid = _routing(inputs, Wg)
    xs = _dispatch(inputs, dest)
    ys = _expert_matmul(eid.reshape(NB), xs, We, be)
    return _combine(ys, dest, w1x, w2x)


# speedup vs baseline: 1.3031x; 1.0503x over previous
import jax
import jax.numpy as jnp
from jax import lax
from jax.experimental import pallas as pl
from jax.experimental.pallas import tpu as pltpu
from jax.experimental.pallas import tpu_sc as plsc

T = 2048
D = 1024
E = 8
K = 2
A = K * T
B = 512
NPAD = A + E * B
NB = NPAD // B
NEG_INF = -1e30

NUM_TILES = 32
TOK_PER_TILE = T // NUM_TILES
CCH = 32


def _routing_body(x_ref, wg_ref, dest_ref, w1x_ref, w2x_ref, eid_ref):
    logits = jax.lax.dot_general(
        wg_ref[...], x_ref[...], (((1,), (1,)), ((), ())),
        preferred_element_type=jnp.float32)
    sub = jax.lax.broadcasted_iota(jnp.int32, (E, T), 0)
    m1 = jnp.max(logits, axis=0, keepdims=True)
    a1 = jnp.min(jnp.where(logits == m1, sub, E), axis=0, keepdims=True)
    masked = jnp.where(sub == a1, NEG_INF, logits)
    m2 = jnp.max(masked, axis=0, keepdims=True)
    a2 = jnp.min(jnp.where(masked == m2, sub, E), axis=0, keepdims=True)
    w1 = 1.0 / (1.0 + jnp.exp(m2 - m1))
    w2 = 1.0 - w1

    ind = (jnp.where(sub == a1, 1.0, 0.0)
           + jnp.where(sub == a2, 1.0, 0.0))
    r = jax.lax.broadcasted_iota(jnp.int32, (T, T), 0)
    c = jax.lax.broadcasted_iota(jnp.int32, (T, T), 1)
    ut = jnp.where(r < c, 1.0, 0.0)
    rank = jax.lax.dot_general(
        ind, ut, (((1,), (0,)), ((), ())),
        preferred_element_type=jnp.float32)
    counts = jnp.sum(ind, axis=1, keepdims=True).astype(jnp.int32)
    padded = ((counts + (B - 1)) // B) * B
    esub = jax.lax.broadcasted_iota(jnp.int32, (E, E), 0)
    ecol = jax.lax.broadcasted_iota(jnp.int32, (E, E), 1)
    ltri = jnp.where(ecol < esub, 1.0, 0.0)
    pstart = jax.lax.dot_general(
        ltri, padded.astype(jnp.float32), (((1,), (0,)), ((), ())),
        preferred_element_type=jnp.float32)

    slot = rank + pstart
    d1 = jnp.sum(jnp.where(sub == a1, slot, 0.0), axis=0, keepdims=True)
    d2 = jnp.sum(jnp.where(sub == a2, slot, 0.0), axis=0, keepdims=True)
    dest_ref[0:1, :] = d1.astype(jnp.int32)
    dest_ref[1:2, :] = d2.astype(jnp.int32)

    ident = jnp.where(r == c, 1.0, 0.0)
    wrows = jnp.concatenate([w1, w2], axis=0)
    wcol = jax.lax.dot_general(
        ident, wrows, (((1,), (1,)), ((), ())),
        preferred_element_type=jnp.float32)
    w1x_ref[...] = jax.lax.broadcast_in_dim(wcol[:, 0:1], (T, 16), (0, 1))
    w2x_ref[...] = jax.lax.broadcast_in_dim(wcol[:, 1:2], (T, 16), (0, 1))

    bstart = jax.lax.broadcasted_iota(jnp.int32, (E, NB), 1) * B
    ge = jnp.where(bstart >= pstart.astype(jnp.int32), 1, 0)
    eid_ref[...] = jnp.sum(ge, axis=0, keepdims=True) - 1


def _routing(x, Wg):
    return pl.pallas_call(
        _routing_body,
        grid=(1,),
        in_specs=[
            pl.BlockSpec((T, D), lambda i: (0, 0)),
            pl.BlockSpec((E, D), lambda i: (0, 0)),
        ],
        out_specs=[
            pl.BlockSpec((K, T), lambda i: (0, 0)),
            pl.BlockSpec((T, 16), lambda i: (0, 0)),
            pl.BlockSpec((T, 16), lambda i: (0, 0)),
            pl.BlockSpec((1, NB), lambda i: (0, 0)),
        ],
        out_shape=[
            jax.ShapeDtypeStruct((K, T), jnp.int32),
            jax.ShapeDtypeStruct((T, 16), jnp.float32),
            jax.ShapeDtypeStruct((T, 16), jnp.float32),
            jax.ShapeDtypeStruct((1, NB), jnp.int32),
        ],
    )(x, Wg)


def _dispatch_body(x_hbm, dest_hbm, xs_hbm, idx0, idx1, rows, sem0, sem1):
    wid = lax.axis_index("s") * 2 + lax.axis_index("c")
    tb = wid * TOK_PER_TILE
    pltpu.sync_copy(dest_hbm.at[0, pl.ds(tb, TOK_PER_TILE)], idx0)
    pltpu.sync_copy(dest_hbm.at[1, pl.ds(tb, TOK_PER_TILE)], idx1)
    pltpu.sync_copy(x_hbm.at[pl.ds(tb, TOK_PER_TILE)], rows)
    c0 = pltpu.async_copy(rows, xs_hbm.at[idx0], sem0)
    c1 = pltpu.async_copy(rows, xs_hbm.at[idx1], sem1)
    c0.wait()
    c1.wait()


def _dispatch(x, dest):
    f = pl.kernel(
        _dispatch_body,
        out_type=jax.ShapeDtypeStruct((NPAD, D), jnp.float32),
        mesh=plsc.VectorSubcoreMesh(core_axis_name="c", subcore_axis_name="s"),
        compiler_params=pltpu.CompilerParams(needs_layout_passes=False),
        scratch_types=[
            pltpu.VMEM((TOK_PER_TILE,), jnp.int32),
            pltpu.VMEM((TOK_PER_TILE,), jnp.int32),
            pltpu.VMEM((TOK_PER_TILE, D), jnp.float32),
            pltpu.SemaphoreType.DMA,
            pltpu.SemaphoreType.DMA,
        ],
    )
    return f(x, dest)


def _expert_body(eid_ref, xs_ref, we_ref, be_ref, ys_ref):
    y = jax.lax.dot_general(
        xs_ref[...].astype(jnp.bfloat16),
        we_ref[0].astype(jnp.bfloat16),
        (((1,), (1,)), ((), ())),
        preferred_element_type=jnp.float32)
    ys_ref[...] = y + be_ref[0]


def _expert_matmul(eid, xs, We, be):
    grid_spec = pltpu.PrefetchScalarGridSpec(
        num_scalar_prefetch=1,
        grid=(NB,),
        in_specs=[
            pl.BlockSpec((B, D), lambda i, eid: (i, 0)),
            pl.BlockSpec((1, D, D), lambda i, eid: (eid[i], 0, 0)),
            pl.BlockSpec((1, 1, D), lambda i, eid: (eid[i], 0, 0)),
        ],
        out_specs=pl.BlockSpec((B, D), lambda i, eid: (i, 0)),
    )
    return pl.pallas_call(
        _expert_body,
        grid_spec=grid_spec,
        out_shape=jax.ShapeDtypeStruct((NPAD, D), jnp.float32),
    )(eid, xs, We, be.reshape(E, 1, D))


def _combine_body(ys_hbm, dest_hbm, w1x_hbm, w2x_hbm, out_hbm,
                  d0, d1, wv1, wv2, buf0, buf1, sem0, sem1):
    wid = lax.axis_index("s") * 2 + lax.axis_index("c")
    tb = wid * TOK_PER_TILE
    pltpu.sync_copy(dest_hbm.at[0, pl.ds(tb, TOK_PER_TILE)], d0)
    pltpu.sync_copy(dest_hbm.at[1, pl.ds(tb, TOK_PER_TILE)], d1)
    pltpu.sync_copy(w1x_hbm.at[pl.ds(tb, TOK_PER_TILE)], wv1)
    pltpu.sync_copy(w2x_hbm.at[pl.ds(tb, TOK_PER_TILE)], wv2)
    for ch in range(TOK_PER_TILE // CCH):
        c0 = pltpu.async_copy(
            ys_hbm.at[d0.at[pl.ds(ch * CCH, CCH)]], buf0, sem0)
        c1 = pltpu.async_copy(
            ys_hbm.at[d1.at[pl.ds(ch * CCH, CCH)]], buf1, sem1)
        c0.wait()
        c1.wait()

        @plsc.parallel_loop(0, CCH * (D // 16), step=1, unroll=8)
        def _fma(j):
            t = j >> 6
            i = j & (D // 16 - 1)
            w1v = wv1[ch * CCH + t, :]
            w2v = wv2[ch * CCH + t, :]
            buf0[t, pl.ds(i * 16, 16)] = (
                w1v * buf0[t, pl.ds(i * 16, 16)]
                + w2v * buf1[t, pl.ds(i * 16, 16)])

        pltpu.sync_copy(buf0, out_hbm.at[pl.ds(tb + ch * CCH, CCH)])


def _combine(ys, dest, w1x, w2x):
    f = pl.kernel(
        _combine_body,
        out_type=jax.ShapeDtypeStruct((T, D), jnp.float32),
        mesh=plsc.VectorSubcoreMesh(core_axis_name="c", subcore_axis_name="s"),
        compiler_params=pltpu.CompilerParams(needs_layout_passes=False),
        scratch_types=[
            pltpu.VMEM((TOK_PER_TILE,), jnp.int32),
            pltpu.VMEM((TOK_PER_TILE,), jnp.int32),
            pltpu.VMEM((TOK_PER_TILE, 16), jnp.float32),
            pltpu.VMEM((TOK_PER_TILE, 16), jnp.float32),
            pltpu.VMEM((CCH, D), jnp.float32),
            pltpu.VMEM((CCH, D), jnp.float32),
            pltpu.SemaphoreType.DMA,
            pltpu.SemaphoreType.DMA,
        ],
    )
    return f(ys, dest, w1x, w2x)


def kernel(inputs, Wg, We, be):
    dest, w1x, w2x, eid = _routing(inputs, Wg)
    xs = _dispatch(inputs, dest)
    ys = _expert_matmul(eid.reshape(NB), xs, We, be)
    return _combine(ys, dest, w1x, w2x)
